# baseline (speedup 1.0000x reference)
import functools

import jax
import jax.numpy as jnp
from jax.experimental import pallas as pl
from jax.experimental.pallas import tpu as pltpu

_VMEM_LIMIT = 48 * 1024 * 1024


def _largest_divisor(n, cap):
    for c in range(min(n, cap), 0, -1):
        if n % c == 0:
            return c
    return 1


def _dual_lstm_kernel(xf_s, xb_s, xf_p, xb_p, w_ih_ref, w_hh_ref, b_ref,
                      out_f_ref, out_b_ref,
                      h01_scr, gxf_scr, gxb_scr,
                      hf_scr, cf_scr, hb_scr, cb_scr,
                      *, tc, nc, in_w):
    l = pl.program_id(0)
    c = pl.program_id(1)
    B, H = hf_scr.shape
    four_h = b_ref.shape[-1]

    def finish(acc, d, rows):
        return ((acc + b_ref[0, d]).reshape(rows, B, four_h)
                .astype(jnp.bfloat16))

    def proj_x(x_ref, d):
        xp = x_ref[...].reshape(tc * B, in_w)
        acc = jnp.dot(xp, w_ih_ref[0, d],
                      preferred_element_type=jnp.float32)
        return finish(acc, d, tc)

    def proj_h(d, tstart):
        hfp = h01_scr[0, pl.ds(tstart, tc)].reshape(tc * B, H)
        hbp = h01_scr[1, pl.ds(tstart, tc)].reshape(tc * B, H)
        acc = (jnp.dot(hfp, w_ih_ref[0, d, pl.ds(0, H)],
                       preferred_element_type=jnp.float32)
               + jnp.dot(hbp, w_ih_ref[0, d, pl.ds(H, H)],
                         preferred_element_type=jnp.float32))
        return finish(acc, d, tc)

    @pl.when(c == 0)
    def _():
        hf_scr[...] = jnp.zeros_like(hf_scr)
        cf_scr[...] = jnp.zeros_like(cf_scr)
        hb_scr[...] = jnp.zeros_like(hb_scr)
        cb_scr[...] = jnp.zeros_like(cb_scr)

    @pl.when(jnp.logical_and(l == 0, c == 0))
    def _():
        gxf_scr[...] = proj_x(xf_p, 0)
        gxb_scr[...] = proj_x(xb_p, 1)

    @pl.when(jnp.logical_and(l == 1, c == 0))
    def _():
        gxf_scr[...] = proj_h(0, 0)
        gxb_scr[...] = proj_h(1, (nc - 1) * tc)

    def lstm_step(gates, c_prev):
        sig = jax.nn.sigmoid(gates[:, :3 * H])
        i_g = sig[:, 0 * H:1 * H]
        f_g = sig[:, 1 * H:2 * H]
        o_g = sig[:, 2 * H:3 * H]
        g_g = jnp.tanh(gates[:, 3 * H:])
        c_new = f_g * c_prev + i_g * g_g
        h_new = (o_g * jnp.tanh(c_new)).astype(jnp.bfloat16)
        return h_new, c_new

    def rec_step(i, ib):
        g_f = gxf_scr[pl.ds(i, 1)][0] + jnp.dot(
            hf_scr[...], w_hh_ref[0, 0], preferred_element_type=jnp.float32)
        h_f, c_f = lstm_step(g_f, cf_scr[...])
        hf_scr[...] = h_f
        cf_scr[...] = c_f

        g_b = gxb_scr[pl.ds(ib, 1)][0] + jnp.dot(
            hb_scr[...], w_hh_ref[0, 1], preferred_element_type=jnp.float32)
        h_b, c_b = lstm_step(g_b, cb_scr[...])
        hb_scr[...] = h_b
        cb_scr[...] = c_b

        @pl.when(l == 0)
        def _():
            h01_scr[0, pl.ds(c * tc + i, 1)] = h_f[None]
            h01_scr[1, pl.ds((nc - 1 - c) * tc + ib, 1)] = h_b[None]

        out_f_ref[pl.ds(i, 1)] = h_f[None]
        out_b_ref[pl.ds(ib, 1)] = h_b[None]

    for k in range(tc):
        rec_step(k, tc - 1 - k)

    @pl.when(l == 0)
    def _():
        gxf_scr[...] = proj_x(xf_s, 0)
        gxb_scr[...] = proj_x(xb_s, 1)

    @pl.when(l == 1)
    def _():
        gxf_scr[...] = proj_h(0, jnp.minimum(c + 1, nc - 1) * tc)
        gxb_scr[...] = proj_h(1, jnp.maximum(nc - 2 - c, 0) * tc)


def _dual_bilstm(x_tm, w_ih_all, w_hh_all, b_all, *, time_chunk=32):
    T, B, I = x_tm.shape
    H = int(w_hh_all.shape[2])
    four_h = int(w_hh_all.shape[-1])
    tc = _largest_divisor(T, time_chunk)
    nc = T // tc

    def xspec(idx_fn):
        return pl.BlockSpec((tc, B, I), lambda l, c, f=idx_fn: (f(l, c), 0, 0))

    sf = lambda l, c: jnp.where(l == 0, jnp.minimum(c + 1, nc - 1), nc - 1)
    sb = lambda l, c: jnp.where(l == 0, jnp.maximum(nc - 2 - c, 0), 0)
    in_specs = [
        xspec(sf), xspec(sb),
        xspec(lambda l, c: 0), xspec(lambda l, c: nc - 1),
        pl.BlockSpec((1, 2, I, four_h), lambda l, c: (l, 0, 0, 0)),
        pl.BlockSpec((1, 2, H, four_h), lambda l, c: (l, 0, 0, 0)),
        pl.BlockSpec((1, 2, 1, four_h), lambda l, c: (l, 0, 0, 0)),
    ]

    kernel_fn = functools.partial(_dual_lstm_kernel, tc=tc, nc=nc, in_w=I)
    return pl.pallas_call(
        kernel_fn,
        out_shape=(jax.ShapeDtypeStruct((T, B, H), jnp.bfloat16),
                   jax.ShapeDtypeStruct((T, B, H), jnp.bfloat16)),
        grid_spec=pltpu.PrefetchScalarGridSpec(
            num_scalar_prefetch=0,
            grid=(2, nc),
            in_specs=in_specs,
            out_specs=[
                pl.BlockSpec((tc, B, H), lambda l, c: (c * l, 0, 0)),
                pl.BlockSpec((tc, B, H),
                             lambda l, c: (nc - 1 - c * l, 0, 0)),
            ],
            scratch_shapes=[
                pltpu.VMEM((2, T, B, H), jnp.bfloat16),
                pltpu.VMEM((tc, B, four_h), jnp.bfloat16),
                pltpu.VMEM((tc, B, four_h), jnp.bfloat16),
                pltpu.VMEM((B, H), jnp.bfloat16),
                pltpu.VMEM((B, H), jnp.float32),
                pltpu.VMEM((B, H), jnp.bfloat16),
                pltpu.VMEM((B, H), jnp.float32),
            ]),
        compiler_params=pltpu.CompilerParams(
            dimension_semantics=("arbitrary", "arbitrary"),
            vmem_limit_bytes=_VMEM_LIMIT),
    )(x_tm, x_tm, x_tm, x_tm, w_ih_all, w_hh_all, b_all)


def _heads_kernel(hf_ref, hb_ref, w_ref, b_ref, lstm_ref, frame_ref, video_ref):
    c = pl.program_id(0)
    tt, B, H = hf_ref.shape
    two_c = w_ref.shape[-1]
    C = two_c // 2

    h_cat = jnp.concatenate([hf_ref[...], hb_ref[...]], axis=-1)
    lstm_ref[...] = jnp.transpose(h_cat, (1, 0, 2)).astype(jnp.float32)

    scores = (jnp.dot(h_cat.reshape(tt * B, 2 * H), w_ref[...],
                      preferred_element_type=jnp.float32)
              + b_ref[...]).reshape(tt, B, two_c)
    frame_ref[...] = jnp.transpose(scores[:, :, :C], (1, 0, 2))

    @pl.when(c == pl.num_programs(0) - 1)
    def _():
        video_ref[...] = scores[tt - 1, :, C:]


def _fused_heads(h_f, h_b, w_heads, b_heads, *, time_tile=64):
    T, B, H = h_f.shape
    two_c = int(w_heads.shape[-1])
    C = two_c // 2
    tt = _largest_divisor(T, time_tile)
    return pl.pallas_call(
        _heads_kernel,
        out_shape=(jax.ShapeDtypeStruct((B, T, 2 * H), jnp.float32),
                   jax.ShapeDtypeStruct((B, T, C), jnp.float32),
                   jax.ShapeDtypeStruct((B, C), jnp.float32)),
        grid_spec=pltpu.PrefetchScalarGridSpec(
            num_scalar_prefetch=0,
            grid=(T // tt,),
            in_specs=[
                pl.BlockSpec((tt, B, H), lambda c: (c, 0, 0)),
                pl.BlockSpec((tt, B, H), lambda c: (c, 0, 0)),
                pl.BlockSpec((2 * H, two_c), lambda c: (0, 0)),
                pl.BlockSpec((1, two_c), lambda c: (0, 0)),
            ],
            out_specs=[pl.BlockSpec((B, tt, 2 * H), lambda c: (0, c, 0)),
                       pl.BlockSpec((B, tt, C), lambda c: (0, c, 0)),
                       pl.BlockSpec((B, C), lambda c: (0, 0))]),
        compiler_params=pltpu.CompilerParams(
            dimension_semantics=("arbitrary",),
            vmem_limit_bytes=_VMEM_LIMIT),
    )(h_f, h_b, w_heads, b_heads)


def kernel(l0_w_ih, l0_w_hh, l0_b, l1_w_ih, l1_w_hh, l1_b, w_heads, b_heads, x):
    x_tm = jnp.transpose(x, (1, 0, 2)).astype(jnp.bfloat16)
    w_ih_all = jnp.stack([l0_w_ih, l1_w_ih])
    w_hh_all = jnp.stack([l0_w_hh, l1_w_hh])
    b_all = jnp.stack([l0_b, l1_b])
    f1, b1 = _dual_bilstm(x_tm, w_ih_all, w_hh_all, b_all)
    lstm_out, frame_out, video_out = _fused_heads(f1, b1, w_heads, b_heads)
    return video_out, frame_out, lstm_out


# speedup vs baseline: 1.3061x; 1.3061x over previous
import functools

import jax
import jax.numpy as jnp
from jax.experimental import pallas as pl
from jax.experimental.pallas import tpu as pltpu

_VMEM_LIMIT = 48 * 1024 * 1024


def _largest_divisor(n, cap):
    for c in range(min(n, cap), 0, -1):
        if n % c == 0:
            return c
    return 1


def _dual_lstm_kernel(xf_s, xb_s, xf_p, xb_p, w_ih_ref, w_hh_ref, b_ref,
                      out_f_ref, out_b_ref,
                      h01_scr, gxf_scr, gxb_scr,
                      hf_scr, cf_scr, hb_scr, cb_scr,
                      *, tc, nc, in_w):
    l = pl.program_id(0)
    c = pl.program_id(1)
    B, H = hf_scr.shape
    four_h = b_ref.shape[-1]

    def finish(acc, d, rows):
        return ((acc + b_ref[0, d]).reshape(rows, B, four_h)
                .astype(jnp.bfloat16))

    def proj_x(x_ref, d):
        xp = x_ref[...].reshape(tc * B, in_w)
        acc = jnp.dot(xp, w_ih_ref[0, d],
                      preferred_element_type=jnp.float32)
        return finish(acc, d, tc)

    def proj_h(d, tstart):
        hfp = h01_scr[0, pl.ds(tstart, tc)].reshape(tc * B, H)
        hbp = h01_scr[1, pl.ds(tstart, tc)].reshape(tc * B, H)
        acc = (jnp.dot(hfp, w_ih_ref[0, d, pl.ds(0, H)],
                       preferred_element_type=jnp.float32)
               + jnp.dot(hbp, w_ih_ref[0, d, pl.ds(H, H)],
                         preferred_element_type=jnp.float32))
        return finish(acc, d, tc)

    @pl.when(c == 0)
    def _():
        hf_scr[...] = jnp.zeros_like(hf_scr)
        cf_scr[...] = jnp.zeros_like(cf_scr)
        hb_scr[...] = jnp.zeros_like(hb_scr)
        cb_scr[...] = jnp.zeros_like(cb_scr)

    @pl.when(jnp.logical_and(l == 0, c == 0))
    def _():
        gxf_scr[...] = proj_x(xf_p, 0)
        gxb_scr[...] = proj_x(xb_p, 1)

    @pl.when(jnp.logical_and(l == 1, c == 0))
    def _():
        gxf_scr[...] = proj_h(0, 0)
        gxb_scr[...] = proj_h(1, (nc - 1) * tc)

    def lstm_step(gates, c_prev):
        sig = jax.nn.sigmoid(gates[:, :3 * H])
        i_g = sig[:, 0 * H:1 * H]
        f_g = sig[:, 1 * H:2 * H]
        o_g = sig[:, 2 * H:3 * H]
        g_g = jnp.tanh(gates[:, 3 * H:])
        c_new = f_g * c_prev + i_g * g_g
        h_new = (o_g * jnp.tanh(c_new)).astype(jnp.bfloat16)
        return h_new, c_new

    def rec_step(i, ib):
        g_f = gxf_scr[pl.ds(i, 1)][0] + jnp.dot(
            hf_scr[...], w_hh_ref[0, 0], preferred_element_type=jnp.float32)
        h_f, c_f = lstm_step(g_f, cf_scr[...])
        hf_scr[...] = h_f
        cf_scr[...] = c_f

        g_b = gxb_scr[pl.ds(ib, 1)][0] + jnp.dot(
            hb_scr[...], w_hh_ref[0, 1], preferred_element_type=jnp.float32)
        h_b, c_b = lstm_step(g_b, cb_scr[...])
        hb_scr[...] = h_b
        cb_scr[...] = c_b

        out_f_ref[pl.ds(i, 1)] = h_f[None]
        out_b_ref[pl.ds(ib, 1)] = h_b[None]

    for k in range(tc):
        rec_step(k, tc - 1 - k)

    @pl.when(l == 0)
    def _():
        h01_scr[0, pl.ds(c * tc, tc)] = out_f_ref[...]
        h01_scr[1, pl.ds((nc - 1 - c) * tc, tc)] = out_b_ref[...]
        gxf_scr[...] = proj_x(xf_s, 0)
        gxb_scr[...] = proj_x(xb_s, 1)

    @pl.when(l == 1)
    def _():
        gxf_scr[...] = proj_h(0, jnp.minimum(c + 1, nc - 1) * tc)
        gxb_scr[...] = proj_h(1, jnp.maximum(nc - 2 - c, 0) * tc)


def _dual_bilstm(x_tm, w_ih_all, w_hh_all, b_all, *, time_chunk=32):
    T, B, I = x_tm.shape
    H = int(w_hh_all.shape[2])
    four_h = int(w_hh_all.shape[-1])
    tc = _largest_divisor(T, time_chunk)
    nc = T // tc

    def xspec(idx_fn):
        return pl.BlockSpec((tc, B, I), lambda l, c, f=idx_fn: (f(l, c), 0, 0))

    sf = lambda l, c: jnp.where(l == 0, jnp.minimum(c + 1, nc - 1), nc - 1)
    sb = lambda l, c: jnp.where(l == 0, jnp.maximum(nc - 2 - c, 0), 0)
    in_specs = [
        xspec(sf), xspec(sb),
        xspec(lambda l, c: 0), xspec(lambda l, c: nc - 1),
        pl.BlockSpec((1, 2, I, four_h), lambda l, c: (l, 0, 0, 0)),
        pl.BlockSpec((1, 2, H, four_h), lambda l, c: (l, 0, 0, 0)),
        pl.BlockSpec((1, 2, 1, four_h), lambda l, c: (l, 0, 0, 0)),
    ]

    kernel_fn = functools.partial(_dual_lstm_kernel, tc=tc, nc=nc, in_w=I)
    return pl.pallas_call(
        kernel_fn,
        out_shape=(jax.ShapeDtypeStruct((T, B, H), jnp.bfloat16),
                   jax.ShapeDtypeStruct((T, B, H), jnp.bfloat16)),
        grid_spec=pltpu.PrefetchScalarGridSpec(
            num_scalar_prefetch=0,
            grid=(2, nc),
            in_specs=in_specs,
            out_specs=[
                pl.BlockSpec((tc, B, H), lambda l, c: (c * l, 0, 0)),
                pl.BlockSpec((tc, B, H),
                             lambda l, c: (nc - 1 - c * l, 0, 0)),
            ],
            scratch_shapes=[
                pltpu.VMEM((2, T, B, H), jnp.bfloat16),
                pltpu.VMEM((tc, B, four_h), jnp.bfloat16),
                pltpu.VMEM((tc, B, four_h), jnp.bfloat16),
                pltpu.VMEM((B, H), jnp.bfloat16),
                pltpu.VMEM((B, H), jnp.float32),
                pltpu.VMEM((B, H), jnp.bfloat16),
                pltpu.VMEM((B, H), jnp.float32),
            ]),
        compiler_params=pltpu.CompilerParams(
            dimension_semantics=("arbitrary", "arbitrary"),
            vmem_limit_bytes=_VMEM_LIMIT),
    )(x_tm, x_tm, x_tm, x_tm, w_ih_all, w_hh_all, b_all)


def _heads_kernel(hf_ref, hb_ref, w_ref, b_ref, lstm_ref, frame_ref, video_ref):
    c = pl.program_id(0)
    tt, B, H = hf_ref.shape
    two_c = w_ref.shape[-1]
    C = two_c // 2

    h_cat = jnp.concatenate([hf_ref[...], hb_ref[...]], axis=-1)
    lstm_ref[...] = jnp.transpose(h_cat, (1, 0, 2)).astype(jnp.float32)

    scores = (jnp.dot(h_cat.reshape(tt * B, 2 * H), w_ref[...],
                      preferred_element_type=jnp.float32)
              + b_ref[...]).reshape(tt, B, two_c)
    frame_ref[...] = jnp.transpose(scores[:, :, :C], (1, 0, 2))

    @pl.when(c == pl.num_programs(0) - 1)
    def _():
        video_ref[...] = scores[tt - 1, :, C:]


def _fused_heads(h_f, h_b, w_heads, b_heads, *, time_tile=64):
    T, B, H = h_f.shape
    two_c = int(w_heads.shape[-1])
    C = two_c // 2
    tt = _largest_divisor(T, time_tile)
    return pl.pallas_call(
        _heads_kernel,
        out_shape=(jax.ShapeDtypeStruct((B, T, 2 * H), jnp.float32),
                   jax.ShapeDtypeStruct((B, T, C), jnp.float32),
                   jax.ShapeDtypeStruct((B, C), jnp.float32)),
        grid_spec=pltpu.PrefetchScalarGridSpec(
            num_scalar_prefetch=0,
            grid=(T // tt,),
            in_specs=[
                pl.BlockSpec((tt, B, H), lambda c: (c, 0, 0)),
                pl.BlockSpec((tt, B, H), lambda c: (c, 0, 0)),
                pl.BlockSpec((2 * H, two_c), lambda c: (0, 0)),
                pl.BlockSpec((1, two_c), lambda c: (0, 0)),
            ],
            out_specs=[pl.BlockSpec((B, tt, 2 * H), lambda c: (0, c, 0)),
                       pl.BlockSpec((B, tt, C), lambda c: (0, c, 0)),
                       pl.BlockSpec((B, C), lambda c: (0, 0))]),
        compiler_params=pltpu.CompilerParams(
            dimension_semantics=("arbitrary",),
            vmem_limit_bytes=_VMEM_LIMIT),
    )(h_f, h_b, w_heads, b_heads)


def kernel(l0_w_ih, l0_w_hh, l0_b, l1_w_ih, l1_w_hh, l1_b, w_heads, b_heads, x):
    x_tm = jnp.transpose(x, (1, 0, 2)).astype(jnp.bfloat16)
    w_ih_all = jnp.stack([l0_w_ih, l1_w_ih])
    w_hh_all = jnp.stack([l0_w_hh, l1_w_hh])
    b_all = jnp.stack([l0_b, l1_b])
    f1, b1 = _dual_bilstm(x_tm, w_ih_all, w_hh_all, b_all)
    lstm_out, frame_out, video_out = _fused_heads(f1, b1, w_heads, b_heads)
    return video_out, frame_out, lstm_out


# speedup vs baseline: 1.3274x; 1.0163x over previous
import functools

import jax
import jax.numpy as jnp
from jax.experimental import pallas as pl
from jax.experimental.pallas import tpu as pltpu

_VMEM_LIMIT = 48 * 1024 * 1024


def _largest_divisor(n, cap):
    for c in range(min(n, cap), 0, -1):
        if n % c == 0:
            return c
    return 1


def _bilstm_kernel(*refs, n_parts, part_widths, tc, unroll):
    xf_s = refs[:n_parts]
    xb_s = refs[n_parts:2 * n_parts]
    xf_p = refs[2 * n_parts:3 * n_parts]
    xb_p = refs[3 * n_parts:4 * n_parts]
    (w_ih_ref, w_hh_ref, b_ref, out_f_ref, out_b_ref,
     gxf_scr, gxb_scr, hf_scr, cf_scr, hb_scr, cb_scr) = refs[4 * n_parts:]

    c = pl.program_id(0)
    B, H = hf_scr.shape
    four_h = b_ref.shape[-1]

    def project_chunk(x_refs, d):
        acc = None
        off = 0
        for p, w in enumerate(part_widths):
            xp = x_refs[p][...].reshape(tc * B, w)
            dot = jnp.dot(xp, w_ih_ref[d, pl.ds(off, w)],
                          preferred_element_type=jnp.float32)
            acc = dot if acc is None else acc + dot
            off += w
        return (acc + b_ref[d]).reshape(tc, B, four_h).astype(jnp.bfloat16)

    @pl.when(c == 0)
    def _():
        hf_scr[...] = jnp.zeros_like(hf_scr)
        cf_scr[...] = jnp.zeros_like(cf_scr)
        hb_scr[...] = jnp.zeros_like(hb_scr)
        cb_scr[...] = jnp.zeros_like(cb_scr)
        gxf_scr[...] = project_chunk(xf_p, 0)
        gxb_scr[...] = project_chunk(xb_p, 1)

    def project_slice(x_refs, d, start, rows):
        acc = None
        off = 0
        for p, w in enumerate(part_widths):
            xr = x_refs[p][pl.ds(start, rows)].reshape(rows * B, w)
            dot = jnp.dot(xr, w_ih_ref[d, pl.ds(off, w)],
                          preferred_element_type=jnp.float32)
            acc = dot if acc is None else acc + dot
            off += w
        return (acc + b_ref[d]).reshape(rows, B, four_h).astype(jnp.bfloat16)

    def lstm_step(gates, c_prev):
        sig = jax.nn.sigmoid(gates[:, :3 * H])
        i_g = sig[:, 0 * H:1 * H]
        f_g = sig[:, 1 * H:2 * H]
        o_g = sig[:, 2 * H:3 * H]
        g_g = jnp.tanh(gates[:, 3 * H:])
        c_new = f_g * c_prev + i_g * g_g
        h_new = (o_g * jnp.tanh(c_new)).astype(jnp.bfloat16)
        return h_new, c_new

    def rec_step(i, ib):
        g_f = gxf_scr[pl.ds(i, 1)][0] + jnp.dot(
            hf_scr[...], w_hh_ref[0], preferred_element_type=jnp.float32)
        h_f, c_f = lstm_step(g_f, cf_scr[...])
        hf_scr[...] = h_f
        cf_scr[...] = c_f
        out_f_ref[pl.ds(i, 1)] = h_f[None]

        g_b = gxb_scr[pl.ds(ib, 1)][0] + jnp.dot(
            hb_scr[...], w_hh_ref[1], preferred_element_type=jnp.float32)
        h_b, c_b = lstm_step(g_b, cb_scr[...])
        hb_scr[...] = h_b
        cb_scr[...] = c_b
        out_b_ref[pl.ds(ib, 1)] = h_b[None]

    n_groups = tc // unroll

    def group(j, carry):
        base = j * unroll
        for k in range(unroll):
            rec_step(base + k, tc - 1 - (base + k))
        gxf_scr[pl.ds(base, unroll)] = project_slice(xf_s, 0, base, unroll)
        bbase = tc - base - unroll
        gxb_scr[pl.ds(bbase, unroll)] = project_slice(xb_s, 1, bbase, unroll)
        return carry

    jax.lax.fori_loop(0, n_groups, group, 0)


def _bilstm_layer(parts, w_ih, w_hh, b, *, time_chunk=32, unroll=32):
    T, B = parts[0].shape[0], parts[0].shape[1]
    part_widths = tuple(int(p.shape[-1]) for p in parts)
    n_parts = len(parts)
    H = int(w_hh.shape[1])
    four_h = int(w_hh.shape[-1])
    tc = _largest_divisor(T, time_chunk)
    nc = T // tc

    def spec(idx_fn, w):
        return pl.BlockSpec((tc, B, w), lambda c, f=idx_fn: (f(c), 0, 0))

    nxt_f = lambda c: jnp.minimum(c + 1, nc - 1)
    nxt_b = lambda c: jnp.maximum(nc - 2 - c, 0)
    in_specs = (
        [spec(nxt_f, w) for w in part_widths]
        + [spec(nxt_b, w) for w in part_widths]
        + [spec(lambda c: 0, w) for w in part_widths]
        + [spec(lambda c: nc - 1, w) for w in part_widths]
        + [pl.BlockSpec((2, sum(part_widths), four_h), lambda c: (0, 0, 0)),
           pl.BlockSpec((2, H, four_h), lambda c: (0, 0, 0)),
           pl.BlockSpec((2, 1, four_h), lambda c: (0, 0, 0))])

    kernel_fn = functools.partial(
        _bilstm_kernel, n_parts=n_parts, part_widths=part_widths,
        tc=tc, unroll=_largest_divisor(tc, unroll))
    return pl.pallas_call(
        kernel_fn,
        out_shape=(jax.ShapeDtypeStruct((T, B, H), jnp.bfloat16),
                   jax.ShapeDtypeStruct((T, B, H), jnp.bfloat16)),
        grid_spec=pltpu.PrefetchScalarGridSpec(
            num_scalar_prefetch=0,
            grid=(nc,),
            in_specs=in_specs,
            out_specs=[pl.BlockSpec((tc, B, H), lambda c: (c, 0, 0)),
                       pl.BlockSpec((tc, B, H), lambda c: (nc - 1 - c, 0, 0))],
            scratch_shapes=[
                pltpu.VMEM((tc, B, four_h), jnp.bfloat16),
                pltpu.VMEM((tc, B, four_h), jnp.bfloat16),
                pltpu.VMEM((B, H), jnp.bfloat16),
                pltpu.VMEM((B, H), jnp.float32),
                pltpu.VMEM((B, H), jnp.bfloat16),
                pltpu.VMEM((B, H), jnp.float32),
            ]),
        compiler_params=pltpu.CompilerParams(
            dimension_semantics=("arbitrary",),
            vmem_limit_bytes=_VMEM_LIMIT),
    )(*(list(parts) * 4), w_ih, w_hh, b)


def _heads_kernel(hf_ref, hb_ref, w_ref, b_ref, lstm_ref, frame_ref, video_ref):
    c = pl.program_id(0)
    tt, B, H = hf_ref.shape
    two_c = w_ref.shape[-1]
    C = two_c // 2

    h_cat = jnp.concatenate([hf_ref[...], hb_ref[...]], axis=-1)
    lstm_ref[...] = jnp.transpose(h_cat, (1, 0, 2)).astype(jnp.float32)

    scores = (jnp.dot(h_cat.reshape(tt * B, 2 * H), w_ref[...],
                      preferred_element_type=jnp.float32)
              + b_ref[...]).reshape(tt, B, two_c)
    frame_ref[...] = jnp.transpose(scores[:, :, :C], (1, 0, 2))

    @pl.when(c == pl.num_programs(0) - 1)
    def _():
        video_ref[...] = scores[tt - 1, :, C:]


def _fused_heads(h_f, h_b, w_heads, b_heads, *, time_tile=64):
    T, B, H = h_f.shape
    two_c = int(w_heads.shape[-1])
    C = two_c // 2
    tt = _largest_divisor(T, time_tile)
    return pl.pallas_call(
        _heads_kernel,
        out_shape=(jax.ShapeDtypeStruct((B, T, 2 * H), jnp.float32),
                   jax.ShapeDtypeStruct((B, T, C), jnp.float32),
                   jax.ShapeDtypeStruct((B, C), jnp.float32)),
        grid_spec=pltpu.PrefetchScalarGridSpec(
            num_scalar_prefetch=0,
            grid=(T // tt,),
            in_specs=[
                pl.BlockSpec((tt, B, H), lambda c: (c, 0, 0)),
                pl.BlockSpec((tt, B, H), lambda c: (c, 0, 0)),
                pl.BlockSpec((2 * H, two_c), lambda c: (0, 0)),
                pl.BlockSpec((1, two_c), lambda c: (0, 0)),
            ],
            out_specs=[pl.BlockSpec((B, tt, 2 * H), lambda c: (0, c, 0)),
                       pl.BlockSpec((B, tt, C), lambda c: (0, c, 0)),
                       pl.BlockSpec((B, C), lambda c: (0, 0))]),
        compiler_params=pltpu.CompilerParams(
            dimension_semantics=("arbitrary",),
            vmem_limit_bytes=_VMEM_LIMIT),
    )(h_f, h_b, w_heads, b_heads)


def kernel(l0_w_ih, l0_w_hh, l0_b, l1_w_ih, l1_w_hh, l1_b, w_heads, b_heads, x):
    x_tm = jnp.transpose(x, (1, 0, 2)).astype(jnp.bfloat16)
    f0, b0 = _bilstm_layer([x_tm], l0_w_ih, l0_w_hh, l0_b)
    f1, b1 = _bilstm_layer([f0, b0], l1_w_ih, l1_w_hh, l1_b)
    lstm_out, frame_out, video_out = _fused_heads(f1, b1, w_heads, b_heads)
    return video_out, frame_out, lstm_out


# speedup vs baseline: 1.3290x; 1.0012x over previous
import functools

import jax
import jax.numpy as jnp
from jax.experimental import pallas as pl
from jax.experimental.pallas import tpu as pltpu

_VMEM_LIMIT = 48 * 1024 * 1024


def _largest_divisor(n, cap):
    for c in range(min(n, cap), 0, -1):
        if n % c == 0:
            return c
    return 1


def _bilstm_kernel(*refs, n_parts, part_widths, tc, unroll):
    xf_s = refs[:n_parts]
    xb_s = refs[n_parts:2 * n_parts]
    xf_p = refs[2 * n_parts:3 * n_parts]
    xb_p = refs[3 * n_parts:4 * n_parts]
    (w_ih_ref, w_hh_ref, b_ref, out_f_ref, out_b_ref,
     gxf_scr, gxb_scr, hf_scr, cf_scr, hb_scr, cb_scr) = refs[4 * n_parts:]

    c = pl.program_id(0)
    B, H = hf_scr.shape
    four_h = b_ref.shape[-1]

    def project_chunk(x_refs, d):
        acc = None
        off = 0
        for p, w in enumerate(part_widths):
            xp = x_refs[p][...].reshape(tc * B, w)
            dot = jnp.dot(xp, w_ih_ref[d, pl.ds(off, w)],
                          preferred_element_type=jnp.float32)
            acc = dot if acc is None else acc + dot
            off += w
        return (acc + b_ref[d]).reshape(tc, B, four_h).astype(jnp.bfloat16)

    @pl.when(c == 0)
    def _():
        hf_scr[...] = jnp.zeros_like(hf_scr)
        cf_scr[...] = jnp.zeros_like(cf_scr)
        hb_scr[...] = jnp.zeros_like(hb_scr)
        cb_scr[...] = jnp.zeros_like(cb_scr)
        gxf_scr[...] = project_chunk(xf_p, 0)
        gxb_scr[...] = project_chunk(xb_p, 1)

    def project_slice(x_refs, d, start, rows):
        acc = None
        off = 0
        for p, w in enumerate(part_widths):
            xr = x_refs[p][pl.ds(start, rows)].reshape(rows * B, w)
            dot = jnp.dot(xr, w_ih_ref[d, pl.ds(off, w)],
                          preferred_element_type=jnp.float32)
            acc = dot if acc is None else acc + dot
            off += w
        return (acc + b_ref[d]).reshape(rows, B, four_h).astype(jnp.bfloat16)

    def lstm_step(gates, c_prev):
        sig = jax.nn.sigmoid(gates[:, :3 * H])
        i_g = sig[:, 0 * H:1 * H]
        f_g = sig[:, 1 * H:2 * H]
        o_g = sig[:, 2 * H:3 * H]
        g_g = jnp.tanh(gates[:, 3 * H:])
        c_new = f_g * c_prev + i_g * g_g
        h_new = (o_g * jnp.tanh(c_new)).astype(jnp.bfloat16)
        return h_new, c_new

    def rec_step(i, ib):
        g_f = gxf_scr[pl.ds(i, 1)][0] + jnp.dot(
            hf_scr[...], w_hh_ref[0], preferred_element_type=jnp.float32)
        h_f, c_f = lstm_step(g_f, cf_scr[...])
        hf_scr[...] = h_f
        cf_scr[...] = c_f
        out_f_ref[pl.ds(i, 1)] = h_f[None]

        g_b = gxb_scr[pl.ds(ib, 1)][0] + jnp.dot(
            hb_scr[...], w_hh_ref[1], preferred_element_type=jnp.float32)
        h_b, c_b = lstm_step(g_b, cb_scr[...])
        hb_scr[...] = h_b
        cb_scr[...] = c_b
        out_b_ref[pl.ds(ib, 1)] = h_b[None]

    n_groups = tc // unroll

    def group(j, carry):
        base = j * unroll
        for k in range(unroll):
            rec_step(base + k, tc - 1 - (base + k))
        gxf_scr[pl.ds(base, unroll)] = project_slice(xf_s, 0, base, unroll)
        bbase = tc - base - unroll
        gxb_scr[pl.ds(bbase, unroll)] = project_slice(xb_s, 1, bbase, unroll)
        return carry

    jax.lax.fori_loop(0, n_groups, group, 0)


def _bilstm_layer(parts, w_ih, w_hh, b, *, time_chunk=32, unroll=32):
    T, B = parts[0].shape[0], parts[0].shape[1]
    part_widths = tuple(int(p.shape[-1]) for p in parts)
    n_parts = len(parts)
    H = int(w_hh.shape[1])
    four_h = int(w_hh.shape[-1])
    tc = _largest_divisor(T, time_chunk)
    nc = T // tc

    def spec(idx_fn, w):
        return pl.BlockSpec((tc, B, w), lambda c, f=idx_fn: (f(c), 0, 0))

    nxt_f = lambda c: jnp.minimum(c + 1, nc - 1)
    nxt_b = lambda c: jnp.maximum(nc - 2 - c, 0)
    in_specs = (
        [spec(nxt_f, w) for w in part_widths]
        + [spec(nxt_b, w) for w in part_widths]
        + [spec(lambda c: 0, w) for w in part_widths]
        + [spec(lambda c: nc - 1, w) for w in part_widths]
        + [pl.BlockSpec((2, sum(part_widths), four_h), lambda c: (0, 0, 0)),
           pl.BlockSpec((2, H, four_h), lambda c: (0, 0, 0)),
           pl.BlockSpec((2, 1, four_h), lambda c: (0, 0, 0))])

    kernel_fn = functools.partial(
        _bilstm_kernel, n_parts=n_parts, part_widths=part_widths,
        tc=tc, unroll=_largest_divisor(tc, unroll))
    return pl.pallas_call(
        kernel_fn,
        out_shape=(jax.ShapeDtypeStruct((T, B, H), jnp.bfloat16),
                   jax.ShapeDtypeStruct((T, B, H), jnp.bfloat16)),
        grid_spec=pltpu.PrefetchScalarGridSpec(
            num_scalar_prefetch=0,
            grid=(nc,),
            in_specs=in_specs,
            out_specs=[pl.BlockSpec((tc, B, H), lambda c: (c, 0, 0)),
                       pl.BlockSpec((tc, B, H), lambda c: (nc - 1 - c, 0, 0))],
            scratch_shapes=[
                pltpu.VMEM((tc, B, four_h), jnp.bfloat16),
                pltpu.VMEM((tc, B, four_h), jnp.bfloat16),
                pltpu.VMEM((B, H), jnp.bfloat16),
                pltpu.VMEM((B, H), jnp.float32),
                pltpu.VMEM((B, H), jnp.bfloat16),
                pltpu.VMEM((B, H), jnp.float32),
            ]),
        compiler_params=pltpu.CompilerParams(
            dimension_semantics=("arbitrary",),
            vmem_limit_bytes=_VMEM_LIMIT),
    )(*(list(parts) * 4), w_ih, w_hh, b)


def _heads_kernel(hf_ref, hb_ref, w_ref, b_ref, lstm_ref, frame_ref, video_ref):
    c = pl.program_id(0)
    tt, B, H = hf_ref.shape
    two_c = w_ref.shape[-1]
    C = two_c // 2

    h_cat = jnp.concatenate([hf_ref[...], hb_ref[...]], axis=-1)
    lstm_ref[...] = jnp.transpose(h_cat, (1, 0, 2)).astype(jnp.float32)

    scores = (jnp.dot(h_cat.reshape(tt * B, 2 * H), w_ref[...],
                      preferred_element_type=jnp.float32)
              + b_ref[...]).reshape(tt, B, two_c)
    frame_ref[...] = jnp.transpose(scores[:, :, :C], (1, 0, 2))

    @pl.when(c == pl.num_programs(0) - 1)
    def _():
        video_ref[...] = scores[tt - 1, :, C:]


def _fused_heads(h_f, h_b, w_heads, b_heads, *, time_tile=128):
    T, B, H = h_f.shape
    two_c = int(w_heads.shape[-1])
    C = two_c // 2
    tt = _largest_divisor(T, time_tile)
    return pl.pallas_call(
        _heads_kernel,
        out_shape=(jax.ShapeDtypeStruct((B, T, 2 * H), jnp.float32),
                   jax.ShapeDtypeStruct((B, T, C), jnp.float32),
                   jax.ShapeDtypeStruct((B, C), jnp.float32)),
        grid_spec=pltpu.PrefetchScalarGridSpec(
            num_scalar_prefetch=0,
            grid=(T // tt,),
            in_specs=[
                pl.BlockSpec((tt, B, H), lambda c: (c, 0, 0)),
                pl.BlockSpec((tt, B, H), lambda c: (c, 0, 0)),
                pl.BlockSpec((2 * H, two_c), lambda c: (0, 0)),
                pl.BlockSpec((1, two_c), lambda c: (0, 0)),
            ],
            out_specs=[pl.BlockSpec((B, tt, 2 * H), lambda c: (0, c, 0)),
                       pl.BlockSpec((B, tt, C), lambda c: (0, c, 0)),
                       pl.BlockSpec((B, C), lambda c: (0, 0))]),
        compiler_params=pltpu.CompilerParams(
            dimension_semantics=("arbitrary",),
            vmem_limit_bytes=_VMEM_LIMIT),
    )(h_f, h_b, w_heads, b_heads)


def kernel(l0_w_ih, l0_w_hh, l0_b, l1_w_ih, l1_w_hh, l1_b, w_heads, b_heads, x):
    x_tm = jnp.transpose(x, (1, 0, 2)).astype(jnp.bfloat16)
    f0, b0 = _bilstm_layer([x_tm], l0_w_ih, l0_w_hh, l0_b)
    f1, b1 = _bilstm_layer([f0, b0], l1_w_ih, l1_w_hh, l1_b)
    lstm_out, frame_out, video_out = _fused_heads(f1, b1, w_heads, b_heads)
    return video_out, frame_out, lstm_out


# speedup vs baseline: 1.3352x; 1.0046x over previous
import functools

import jax
import jax.numpy as jnp
from jax.experimental import pallas as pl
from jax.experimental.pallas import tpu as pltpu

_VMEM_LIMIT = 48 * 1024 * 1024


def _largest_divisor(n, cap):
    for c in range(min(n, cap), 0, -1):
        if n % c == 0:
            return c
    return 1


def _bilstm_kernel(*refs, n_parts, part_widths, tc, unroll):
    xf_s = refs[:n_parts]
    xb_s = refs[n_parts:2 * n_parts]
    xf_p = refs[2 * n_parts:3 * n_parts]
    xb_p = refs[3 * n_parts:4 * n_parts]
    (w_ih_ref, w_hh_ref, b_ref, out_f_ref, out_b_ref,
     gxf_scr, gxb_scr, hf_scr, cf_scr, hb_scr, cb_scr) = refs[4 * n_parts:]

    c = pl.program_id(0)
    B, H = hf_scr.shape
    four_h = b_ref.shape[-1]

    def project_chunk(x_refs, d):
        acc = None
        off = 0
        for p, w in enumerate(part_widths):
            xp = x_refs[p][...].reshape(tc * B, w)
            dot = jnp.dot(xp, w_ih_ref[d, pl.ds(off, w)],
                          preferred_element_type=jnp.float32)
            acc = dot if acc is None else acc + dot
            off += w
        return (acc + b_ref[d]).reshape(tc, B, four_h).astype(jnp.bfloat16)

    @pl.when(c == 0)
    def _():
        hf_scr[...] = jnp.zeros_like(hf_scr)
        cf_scr[...] = jnp.zeros_like(cf_scr)
        hb_scr[...] = jnp.zeros_like(hb_scr)
        cb_scr[...] = jnp.zeros_like(cb_scr)
        gxf_scr[...] = project_chunk(xf_p, 0)
        gxb_scr[...] = project_chunk(xb_p, 1)

    def project_slice(x_refs, d, start, rows):
        acc = None
        off = 0
        for p, w in enumerate(part_widths):
            xr = x_refs[p][pl.ds(start, rows)].reshape(rows * B, w)
            dot = jnp.dot(xr, w_ih_ref[d, pl.ds(off, w)],
                          preferred_element_type=jnp.float32)
            acc = dot if acc is None else acc + dot
            off += w
        return (acc + b_ref[d]).reshape(rows, B, four_h).astype(jnp.bfloat16)

    def lstm_step(gates, c_prev):
        sig = jax.nn.sigmoid(gates[:, :3 * H])
        i_g = sig[:, 0 * H:1 * H]
        f_g = sig[:, 1 * H:2 * H]
        o_g = sig[:, 2 * H:3 * H]
        g_g = jnp.tanh(gates[:, 3 * H:])
        c_new = f_g * c_prev + i_g * g_g
        h_new = (o_g * jnp.tanh(c_new)).astype(jnp.bfloat16)
        return h_new, c_new

    def rec_step(i, ib):
        g_f = gxf_scr[pl.ds(i, 1)][0] + jnp.dot(
            hf_scr[...], w_hh_ref[0], preferred_element_type=jnp.float32)
        h_f, c_f = lstm_step(g_f, cf_scr[...])
        hf_scr[...] = h_f
        cf_scr[...] = c_f
        out_f_ref[pl.ds(i, 1)] = h_f[None]

        g_b = gxb_scr[pl.ds(ib, 1)][0] + jnp.dot(
            hb_scr[...], w_hh_ref[1], preferred_element_type=jnp.float32)
        h_b, c_b = lstm_step(g_b, cb_scr[...])
        hb_scr[...] = h_b
        cb_scr[...] = c_b
        out_b_ref[pl.ds(ib, 1)] = h_b[None]

    n_groups = tc // unroll

    def group(j, carry):
        base = j * unroll
        for k in range(unroll):
            rec_step(base + k, tc - 1 - (base + k))
        gxf_scr[pl.ds(base, unroll)] = project_slice(xf_s, 0, base, unroll)
        bbase = tc - base - unroll
        gxb_scr[pl.ds(bbase, unroll)] = project_slice(xb_s, 1, bbase, unroll)
        return carry

    jax.lax.fori_loop(0, n_groups, group, 0)


def _bilstm_layer(parts, w_ih, w_hh, b, *, time_chunk=32, unroll=32):
    T, B = parts[0].shape[0], parts[0].shape[1]
    part_widths = tuple(int(p.shape[-1]) for p in parts)
    n_parts = len(parts)
    H = int(w_hh.shape[1])
    four_h = int(w_hh.shape[-1])
    tc = _largest_divisor(T, time_chunk)
    nc = T // tc

    def spec(idx_fn, w):
        return pl.BlockSpec((tc, B, w), lambda c, f=idx_fn: (f(c), 0, 0))

    nxt_f = lambda c: jnp.minimum(c + 1, nc - 1)
    nxt_b = lambda c: jnp.maximum(nc - 2 - c, 0)
    in_specs = (
        [spec(nxt_f, w) for w in part_widths]
        + [spec(nxt_b, w) for w in part_widths]
        + [spec(lambda c: 0, w) for w in part_widths]
        + [spec(lambda c: nc - 1, w) for w in part_widths]
        + [pl.BlockSpec((2, sum(part_widths), four_h), lambda c: (0, 0, 0)),
           pl.BlockSpec((2, H, four_h), lambda c: (0, 0, 0)),
           pl.BlockSpec((2, 1, four_h), lambda c: (0, 0, 0))])

    kernel_fn = functools.partial(
        _bilstm_kernel, n_parts=n_parts, part_widths=part_widths,
        tc=tc, unroll=_largest_divisor(tc, unroll))
    return pl.pallas_call(
        kernel_fn,
        out_shape=(jax.ShapeDtypeStruct((T, B, H), jnp.bfloat16),
                   jax.ShapeDtypeStruct((T, B, H), jnp.bfloat16)),
        grid_spec=pltpu.PrefetchScalarGridSpec(
            num_scalar_prefetch=0,
            grid=(nc,),
            in_specs=in_specs,
            out_specs=[pl.BlockSpec((tc, B, H), lambda c: (c, 0, 0)),
                       pl.BlockSpec((tc, B, H), lambda c: (nc - 1 - c, 0, 0))],
            scratch_shapes=[
                pltpu.VMEM((tc, B, four_h), jnp.bfloat16),
                pltpu.VMEM((tc, B, four_h), jnp.bfloat16),
                pltpu.VMEM((B, H), jnp.bfloat16),
                pltpu.VMEM((B, H), jnp.float32),
                pltpu.VMEM((B, H), jnp.bfloat16),
                pltpu.VMEM((B, H), jnp.float32),
            ]),
        compiler_params=pltpu.CompilerParams(
            dimension_semantics=("arbitrary",),
            vmem_limit_bytes=_VMEM_LIMIT),
    )(*(list(parts) * 4), w_ih, w_hh, b)


def _heads_kernel(hf_ref, hb_ref, w_ref, b_ref, lstm_ref, frame_ref, video_ref):
    c = pl.program_id(0)
    tt, B, H = hf_ref.shape
    two_c = w_ref.shape[-1]
    C = two_c // 2

    h_cat = jnp.concatenate([hf_ref[...], hb_ref[...]], axis=-1)
    lstm_ref[...] = jnp.transpose(h_cat, (1, 0, 2)).astype(jnp.float32)

    scores = (jnp.dot(h_cat.reshape(tt * B, 2 * H), w_ref[...],
                      preferred_element_type=jnp.float32)
              + b_ref[...]).reshape(tt, B, two_c)
    frame_ref[...] = jnp.transpose(scores[:, :, :C], (1, 0, 2))

    @pl.when(c == pl.num_programs(0) - 1)
    def _():
        video_ref[...] = scores[tt - 1, :, C:]


def _fused_heads(h_f, h_b, w_heads, b_heads, *, time_tile=64):
    T, B, H = h_f.shape
    two_c = int(w_heads.shape[-1])
    C = two_c // 2
    tt = _largest_divisor(T, time_tile)
    return pl.pallas_call(
        _heads_kernel,
        out_shape=(jax.ShapeDtypeStruct((B, T, 2 * H), jnp.float32),
                   jax.ShapeDtypeStruct((B, T, C), jnp.float32),
                   jax.ShapeDtypeStruct((B, C), jnp.float32)),
        grid_spec=pltpu.PrefetchScalarGridSpec(
            num_scalar_prefetch=0,
            grid=(T // tt,),
            in_specs=[
                pl.BlockSpec((tt, B, H), lambda c: (c, 0, 0)),
                pl.BlockSpec((tt, B, H), lambda c: (c, 0, 0)),
                pl.BlockSpec((2 * H, two_c), lambda c: (0, 0)),
                pl.BlockSpec((1, two_c), lambda c: (0, 0)),
            ],
            out_specs=[pl.BlockSpec((B, tt, 2 * H), lambda c: (0, c, 0)),
                       pl.BlockSpec((B, tt, C), lambda c: (0, c, 0)),
                       pl.BlockSpec((B, C), lambda c: (0, 0))]),
        compiler_params=pltpu.CompilerParams(
            dimension_semantics=("arbitrary",),
            vmem_limit_bytes=_VMEM_LIMIT),
    )(h_f, h_b, w_heads, b_heads)


def kernel(l0_w_ih, l0_w_hh, l0_b, l1_w_ih, l1_w_hh, l1_b, w_heads, b_heads, x):
    x_tm = jnp.transpose(x, (1, 0, 2)).astype(jnp.bfloat16)
    f0, b0 = _bilstm_layer([x_tm], l0_w_ih, l0_w_hh, l0_b)
    f1, b1 = _bilstm_layer([f0, b0], l1_w_ih, l1_w_hh, l1_b)
    lstm_out, frame_out, video_out = _fused_heads(f1, b1, w_heads, b_heads)
    return video_out, frame_out, lstm_out
